# baseline (speedup 1.0000x reference)
import functools

import jax
import jax.numpy as jnp
from jax import lax
from jax.experimental import pallas as pl
from jax.experimental.pallas import tpu as pltpu
from jax.experimental.pallas import tpu_sc as plsc

EMB_DIM = 32
BATCH = 16384
LANE = 128

_info = plsc.get_sparse_core_info()
NC, NS, L = _info.num_cores, _info.num_subcores, _info.num_lanes
NW = NC * NS
B_PER_W = BATCH // NW
BURST = 8
N_VECS = B_PER_W // L


def _body(user_hbm, item_hbm, tu_hbm, tv_hbm, out_hbm,
          idx_u, idx_i, out_v, *rest):
    u_bufs = rest[:BURST]
    v_bufs = rest[BURST:2 * BURST]
    sem = rest[2 * BURST]

    wid = lax.axis_index("s") * NC + lax.axis_index("c")
    base = wid * B_PER_W

    pltpu.sync_copy(user_hbm.at[pl.ds(base, B_PER_W)], idx_u)
    pltpu.sync_copy(item_hbm.at[pl.ds(base, B_PER_W)], idx_i)

    lanes = lax.iota(jnp.int32, L)
    d_lo = lax.iota(jnp.int32, L)
    d_hi = d_lo + L

    def vec_step(k, carry):
        k0 = pl.multiple_of(k * L, L)
        iu_vec = idx_u[pl.ds(k0, L)]
        ii_vec = idx_i[pl.ds(k0, L)]
        r = jnp.zeros((L,), jnp.float32)
        for half in range(2):
            handles = []
            for t in range(BURST):
                iu = iu_vec[half * BURST + t]
                ii = ii_vec[half * BURST + t]
                cu = pl.multiple_of(
                    lax.shift_left(lax.shift_right_logical(iu, 7), 7), LANE)
                ci = pl.multiple_of(
                    lax.shift_left(lax.shift_right_logical(ii, 7), 7), LANE)
                handles.append(pltpu.async_copy(
                    tu_hbm.at[:, pl.ds(cu, LANE)], u_bufs[t], sem))
                handles.append(pltpu.async_copy(
                    tv_hbm.at[:, pl.ds(ci, LANE)], v_bufs[t], sem))
            for h in handles:
                h.wait()
            for t in range(BURST):
                iu = iu_vec[half * BURST + t]
                ii = ii_vec[half * BURST + t]
                lu = jnp.full((L,), lax.bitwise_and(iu, LANE - 1), jnp.int32)
                li = jnp.full((L,), lax.bitwise_and(ii, LANE - 1), jnp.int32)
                u0 = plsc.load_gather(u_bufs[t], [d_lo, lu])
                u1 = plsc.load_gather(u_bufs[t], [d_hi, lu])
                v0 = plsc.load_gather(v_bufs[t], [d_lo, li])
                v1 = plsc.load_gather(v_bufs[t], [d_hi, li])
                s = jnp.sum(u0 * v0 + u1 * v1)
                r = jnp.where(lanes == half * BURST + t, s, r)
        out_v[pl.ds(k0, L)] = r
        return carry

    lax.fori_loop(0, N_VECS, vec_step, 0)

    pltpu.sync_copy(out_v, out_hbm.at[pl.ds(base, B_PER_W)])


@jax.jit
def _run(user, item, tu, tv):
    mesh = plsc.VectorSubcoreMesh(core_axis_name="c", subcore_axis_name="s")
    f = functools.partial(
        pl.kernel, mesh=mesh,
        out_type=jax.ShapeDtypeStruct((BATCH,), jnp.float32),
        compiler_params=pltpu.CompilerParams(needs_layout_passes=False),
        scratch_types=[
            pltpu.VMEM((B_PER_W,), jnp.int32),
            pltpu.VMEM((B_PER_W,), jnp.int32),
            pltpu.VMEM((B_PER_W,), jnp.float32),
        ] + [pltpu.VMEM((EMB_DIM, LANE), jnp.float32) for _ in range(2 * BURST)]
        + [pltpu.SemaphoreType.DMA],
    )(_body)
    return f(user, item, tu, tv)


def kernel(user, item, user_emb, item_emb):
    return _run(user.astype(jnp.int32), item.astype(jnp.int32),
                user_emb.T, item_emb.T)
